# Initial kernel scaffold; baseline (speedup 1.0000x reference)
import jax
import jax.numpy as jnp
from jax.experimental import pallas as pl


def kernel(x, Wq, bq, Wk, bk, Wv, bv, Wo, bo, g_attn, b_attn, g_ff, b_ff, g_moe, b_moe, Wg, bg, W1, b1, W2, b2):
    raise NotImplementedError("write your pallas kernel here")


# speedup vs baseline: 1.6079x; 1.6079x over previous
import functools

import jax
import jax.numpy as jnp
from jax.experimental import pallas as pl

H = 12
E = 8
TOP_K = 2
LN_EPS = 1e-5


def _ln(xv, g, b):
    mu = jnp.mean(xv, axis=-1, keepdims=True)
    var = jnp.mean((xv - mu) ** 2, axis=-1, keepdims=True)
    return (xv - mu) * jax.lax.rsqrt(var + LN_EPS) * g + b


def _k1_qkv(x_ref, g_ref, b_ref, wq_ref, bq_ref, wk_ref, bk_ref, wv_ref,
            bv_ref, q_ref, k_ref, v_ref):
    a = _ln(x_ref[...], g_ref[...], b_ref[...])
    q_ref[...] = jnp.dot(a, wq_ref[...], preferred_element_type=jnp.float32) + bq_ref[...]
    k_ref[...] = jnp.dot(a, wk_ref[...], preferred_element_type=jnp.float32) + bk_ref[...]
    v_ref[...] = jnp.dot(a, wv_ref[...], preferred_element_type=jnp.float32) + bv_ref[...]


def _k2_attn(q_ref, k_ref, v_ref, o_ref, *, scale, dh):
    n = q_ref.shape[1] // dh
    for j in range(n):
        sl = slice(j * dh, (j + 1) * dh)
        s = jax.lax.dot_general(
            q_ref[:, sl], k_ref[:, sl], (((1,), (1,)), ((), ())),
            preferred_element_type=jnp.float32) * scale
        m = jnp.max(s, axis=1, keepdims=True)
        p = jnp.exp(s - m)
        p = p / jnp.sum(p, axis=1, keepdims=True)
        o_ref[:, sl] = jnp.dot(p, v_ref[:, sl],
                               preferred_element_type=jnp.float32)


def _k3_proj(x_ref, ao_ref, wo_ref, bo_ref, gf_ref, bf_ref, wg_ref, bg_ref,
             x2_ref, inp_ref, logits_ref):
    o = jnp.dot(ao_ref[...], wo_ref[...], preferred_element_type=jnp.float32) + bo_ref[...]
    x2 = x_ref[...] + o
    x2_ref[...] = x2
    inp = _ln(x2, gf_ref[...], bf_ref[...])
    inp_ref[...] = inp
    logits_ref[...] = jnp.dot(inp, wg_ref[...], preferred_element_type=jnp.float32) + bg_ref[...]


def _k4_moe(t_ref, we_ref, w1_ref, b1_ref, w2_ref, b2_ref, core_ref):
    e = pl.program_id(0)

    t = t_ref[...]
    h = jnp.maximum(
        jnp.dot(t, w1_ref[0], preferred_element_type=jnp.float32) + b1_ref[0],
        0.0)
    h2 = jnp.dot(h, w2_ref[0], preferred_element_type=jnp.float32) + b2_ref[0]
    lane = jax.lax.broadcasted_iota(jnp.int32, we_ref.shape, 1)
    wcol = jnp.sum(jnp.where(lane == e, we_ref[...], 0.0), axis=1,
                   keepdims=True)

    @pl.when(e == 0)
    def _():
        core_ref[...] = jnp.zeros_like(core_ref)

    core_ref[...] += wcol * h2


def _k5_final(x2_ref, inp_ref, core_ref, gm_ref, bm_ref, out_ref):
    o2 = _ln(inp_ref[...] + core_ref[...], gm_ref[...], bm_ref[...])
    out_ref[...] = x2_ref[...] + o2


def kernel(x, Wq, bq, Wk, bk, Wv, bv, Wo, bo, g_attn, b_attn, g_ff, b_ff,
           g_moe, b_moe, Wg, bg, W1, b1, W2, b2):
    B, S, D = x.shape
    dh = D // H
    Dff = W1.shape[-1]
    x2d = x.reshape(S, D)
    row = lambda a: a.reshape(1, -1)

    SB = 256
    NS = S // SB

    full = pl.BlockSpec((1, D), lambda i: (0, 0))
    q, k, v = pl.pallas_call(
        _k1_qkv,
        grid=(NS,),
        in_specs=[pl.BlockSpec((SB, D), lambda i: (i, 0)), full, full,
                  pl.BlockSpec((D, D), lambda i: (0, 0)), full,
                  pl.BlockSpec((D, D), lambda i: (0, 0)), full,
                  pl.BlockSpec((D, D), lambda i: (0, 0)), full],
        out_specs=[pl.BlockSpec((SB, D), lambda i: (i, 0))] * 3,
        out_shape=[jax.ShapeDtypeStruct((S, D), jnp.float32)] * 3,
    )(x2d, row(g_attn), row(b_attn), Wq, row(bq), Wk, row(bk), Wv, row(bv))

    HPB = 2
    head = pl.BlockSpec((S, HPB * dh), lambda h: (0, h))
    ao = pl.pallas_call(
        functools.partial(_k2_attn, scale=1.0 / (dh ** 0.5), dh=dh),
        grid=(H // HPB,),
        in_specs=[head, head, head],
        out_specs=head,
        out_shape=jax.ShapeDtypeStruct((S, D), jnp.float32),
    )(q, k, v)

    EP = 128
    Wg_p = jnp.zeros((D, EP), jnp.float32).at[:, :E].set(Wg)
    bg_p = jnp.zeros((1, EP), jnp.float32).at[0, :E].set(bg)
    x2, inp, logits_p = pl.pallas_call(
        _k3_proj,
        grid=(NS,),
        in_specs=[pl.BlockSpec((SB, D), lambda i: (i, 0)),
                  pl.BlockSpec((SB, D), lambda i: (i, 0)),
                  pl.BlockSpec((D, D), lambda i: (0, 0)), full, full, full,
                  pl.BlockSpec((D, EP), lambda i: (0, 0)),
                  pl.BlockSpec((1, EP), lambda i: (0, 0))],
        out_specs=[pl.BlockSpec((SB, D), lambda i: (i, 0)),
                   pl.BlockSpec((SB, D), lambda i: (i, 0)),
                   pl.BlockSpec((SB, EP), lambda i: (i, 0))],
        out_shape=[jax.ShapeDtypeStruct((S, D), jnp.float32)] * 2
        + [jax.ShapeDtypeStruct((S, EP), jnp.float32)],
    )(x2d, ao, Wo, row(bo), row(g_ff), row(b_ff), Wg_p, bg_p)

    logits = logits_p[:, :E]
    topv, topi = jax.lax.top_k(logits, TOP_K)
    scores = jax.nn.softmax(topv, axis=-1)
    we = jnp.sum(
        jnp.where(topi[:, :, None] == jnp.arange(E)[None, None, :],
                  scores[:, :, None], 0.0), axis=1)

    core = pl.pallas_call(
        _k4_moe,
        grid=(E,),
        in_specs=[pl.BlockSpec((S, D), lambda e: (0, 0)),
                  pl.BlockSpec((S, E), lambda e: (0, 0)),
                  pl.BlockSpec((1, D, Dff), lambda e: (e, 0, 0)),
                  pl.BlockSpec((1, 1, Dff), lambda e: (e, 0, 0)),
                  pl.BlockSpec((1, Dff, D), lambda e: (e, 0, 0)),
                  pl.BlockSpec((1, 1, D), lambda e: (e, 0, 0))],
        out_specs=pl.BlockSpec((S, D), lambda e: (0, 0)),
        out_shape=jax.ShapeDtypeStruct((S, D), jnp.float32),
    )(inp, we, W1, b1.reshape(E, 1, Dff), W2, b2.reshape(E, 1, D))

    out = pl.pallas_call(
        _k5_final,
        grid=(NS,),
        in_specs=[pl.BlockSpec((SB, D), lambda i: (i, 0)),
                  pl.BlockSpec((SB, D), lambda i: (i, 0)),
                  pl.BlockSpec((SB, D), lambda i: (i, 0)), full, full],
        out_specs=pl.BlockSpec((SB, D), lambda i: (i, 0)),
        out_shape=jax.ShapeDtypeStruct((S, D), jnp.float32),
    )(x2, inp, core, row(g_moe), row(b_moe))

    return out.reshape(B, S, D)
